# baseline (speedup 1.0000x reference)
import functools

import jax
import jax.numpy as jnp
from jax import lax
from jax.experimental import pallas as pl
from jax.experimental.pallas import tpu as pltpu
from jax.experimental.pallas import tpu_sc as plsc

N_NODES = 10000
N_EDGES = 320000
D_IN = 128
D_HID = 64
D_OUT = 128

NC = 2
NS = 16
NW = NC * NS
N_PAD = 10240
ROWS_PER_TILE = N_PAD // NS
E_PAD = NW * 10240
E_PER_W = E_PAD // NW
K = 64
CHUNKS = E_PER_W // K
CHUNKS_P = CHUNKS + 2
NBUF = 4
TOTAL_CHUNKS = E_PAD // K
CPP = TOTAL_CHUNKS // NS
N0C = 248

_MESH = dict(core_axis_name="c", subcore_axis_name="s")


@functools.partial(
    pl.kernel,
    out_type=jax.ShapeDtypeStruct((NC, N_PAD), jnp.float32),
    mesh=plsc.VectorSubcoreMesh(**_MESH),
    scratch_types=[
        pltpu.VMEM((CHUNKS, K), jnp.int32),
        pltpu.VMEM((K,), jnp.float32),
        pltpu.VMEM((ROWS_PER_TILE,), jnp.float32),
        pltpu.VMEM_SHARED((N_PAD,), jnp.float32),
        pltpu.SemaphoreType.DMA,
    ],
)
def _deg_kernel(dst_hbm, out_hbm, idx_v, ones_v, zero_v, acc_sh, sem):
    c = lax.axis_index("c")
    s = lax.axis_index("s")
    wid = s * NC + c
    for i in range(K // 16):
        ones_v[pl.ds(i * 16, 16)] = jnp.full((16,), 1.0, jnp.float32)
    for i in range(ROWS_PER_TILE // 16):
        zero_v[pl.ds(i * 16, 16)] = jnp.zeros((16,), jnp.float32)
    pltpu.sync_copy(zero_v, acc_sh.at[pl.ds(s * ROWS_PER_TILE, ROWS_PER_TILE)])
    pltpu.sync_copy(dst_hbm.at[wid], idx_v)
    plsc.subcore_barrier()

    def fire(j, carry):
        pltpu.async_copy(ones_v, acc_sh.at[idx_v.at[j]], sem, add=True)
        return carry

    lax.fori_loop(0, CHUNKS, fire, 0)

    def drain(j, carry):
        pltpu.make_async_copy(ones_v, acc_sh.at[idx_v.at[0]], sem).wait()
        return carry

    lax.fori_loop(0, CHUNKS, drain, 0)
    plsc.subcore_barrier()
    pltpu.sync_copy(acc_sh.at[pl.ds(s * ROWS_PER_TILE, ROWS_PER_TILE)],
                    out_hbm.at[c, pl.ds(s * ROWS_PER_TILE, ROWS_PER_TILE)])


def _make_scatter(D):
    @functools.partial(
        pl.kernel,
        out_type=jax.ShapeDtypeStruct((NC, N_PAD, D), jnp.float32),
        mesh=plsc.VectorSubcoreMesh(**_MESH),
        scratch_types=[
            pltpu.VMEM((NBUF, 2, K), jnp.int32),
            pltpu.VMEM((NBUF, K, D), jnp.float32),
            pltpu.VMEM_SHARED((N_PAD, D), jnp.float32),
            pltpu.SemaphoreType.DMA((NBUF,)),
            pltpu.SemaphoreType.DMA((NBUF,)),
            pltpu.SemaphoreType.DMA((NBUF,)),
        ],
    )
    def scatter(g_hbm, eidx_hbm, out_hbm, ibuf, rows, acc_sh, isem, gsem, ssem):
        c = lax.axis_index("c")
        s = lax.axis_index("s")
        start = s * CPP + c * N0C
        cnt = jnp.where(c == 0, N0C, CPP - N0C)

        def zfill(j, carry):
            for t in range(D // 16):
                rows[0, j, pl.ds(t * 16, 16)] = jnp.zeros((16,), jnp.float32)
            return carry

        lax.fori_loop(0, K, zfill, 0)
        for r in range(ROWS_PER_TILE // K):
            pltpu.sync_copy(
                rows.at[0], acc_sh.at[pl.ds(s * ROWS_PER_TILE + r * K, K)])
        plsc.subcore_barrier()

        pltpu.async_copy(eidx_hbm.at[start], ibuf.at[0], isem.at[0])
        pltpu.async_copy(eidx_hbm.at[start + 1], ibuf.at[1], isem.at[1])
        pltpu.make_async_copy(eidx_hbm.at[0], ibuf.at[0], isem.at[0]).wait()
        pltpu.async_copy(g_hbm.at[ibuf.at[0, 0]], rows.at[0], gsem.at[0])

        def step(j, b, with_swait):
            b2 = (b + 2) % NBUF
            b1 = (b + 1) % NBUF
            if with_swait:
                pltpu.make_async_copy(rows.at[b2], acc_sh.at[ibuf.at[b2, 1]],
                                      ssem.at[b2]).wait()
            pltpu.async_copy(eidx_hbm.at[start + j + 2], ibuf.at[b2],
                             isem.at[b2])
            pltpu.make_async_copy(eidx_hbm.at[0], ibuf.at[b1],
                                  isem.at[b1]).wait()
            pltpu.async_copy(g_hbm.at[ibuf.at[b1, 0]], rows.at[b1],
                             gsem.at[b1])
            pltpu.make_async_copy(g_hbm.at[ibuf.at[b, 0]], rows.at[b],
                                  gsem.at[b]).wait()
            pltpu.async_copy(rows.at[b], acc_sh.at[ibuf.at[b, 1]],
                             ssem.at[b], add=True)

        for j0 in range(NBUF):
            step(j0, j0, with_swait=(j0 >= 2))

        def body(jj, carry):
            for b in range(NBUF):
                step(jj * NBUF + b, b, True)
            return carry

        lax.fori_loop(1, cnt // NBUF, body, 0)
        pltpu.make_async_copy(rows.at[2], acc_sh.at[ibuf.at[2, 1]],
                              ssem.at[2]).wait()
        pltpu.make_async_copy(rows.at[3], acc_sh.at[ibuf.at[3, 1]],
                              ssem.at[3]).wait()
        pltpu.make_async_copy(eidx_hbm.at[0], ibuf.at[1], isem.at[1]).wait()
        pltpu.make_async_copy(g_hbm.at[ibuf.at[0, 0]], rows.at[0],
                              gsem.at[0]).wait()
        plsc.subcore_barrier()
        off = s * ROWS_PER_TILE
        pltpu.sync_copy(acc_sh.at[pl.ds(off, ROWS_PER_TILE)],
                        out_hbm.at[c, pl.ds(off, ROWS_PER_TILE)])

    return scatter


_scatter128 = _make_scatter(128)


_BLK = 512
_GRID = (N_PAD // _BLK,)


def _dense1_body(x_ref, w_ref, d0_ref, d1_ref, g_ref, dinv_ref):
    deg = d0_ref[...] + d1_ref[...] + 1.0
    dinv = lax.rsqrt(deg)
    dinv_ref[...] = dinv
    h = jnp.dot(x_ref[...], w_ref[...], preferred_element_type=jnp.float32)
    g_ref[...] = jnp.concatenate(
        [h * dinv, jnp.zeros((_BLK, 128 - D_HID), jnp.float32)], axis=1)


def _dense1(x_p, W1, d0, d1):
    return pl.pallas_call(
        _dense1_body,
        grid=_GRID,
        in_specs=[
            pl.BlockSpec((_BLK, D_IN), lambda i: (i, 0)),
            pl.BlockSpec((D_IN, D_HID), lambda i: (0, 0)),
            pl.BlockSpec((_BLK, 1), lambda i: (i, 0)),
            pl.BlockSpec((_BLK, 1), lambda i: (i, 0)),
        ],
        out_specs=[
            pl.BlockSpec((_BLK, 128), lambda i: (i, 0)),
            pl.BlockSpec((_BLK, 1), lambda i: (i, 0)),
        ],
        out_shape=[
            jax.ShapeDtypeStruct((N_PAD, 128), jnp.float32),
            jax.ShapeDtypeStruct((N_PAD, 1), jnp.float32),
        ],
    )(x_p, W1, d0, d1)


def _dense2_body(s0_ref, s1_ref, g_ref, dinv_ref, b_ref, w_ref, g2_ref):
    dinv = dinv_ref[...]
    y = dinv * (s0_ref[...] + s1_ref[...] + g_ref[...]) + b_ref[...]
    y = jnp.maximum(y, 0.0)
    h2 = jnp.dot(y, w_ref[...], preferred_element_type=jnp.float32)
    g2_ref[...] = h2 * dinv


def _dense2(s0, s1, g1, dinv, b1, W2):
    return pl.pallas_call(
        _dense2_body,
        grid=_GRID,
        in_specs=[
            pl.BlockSpec((_BLK, 128), lambda i: (i, 0)),
            pl.BlockSpec((_BLK, 128), lambda i: (i, 0)),
            pl.BlockSpec((_BLK, 128), lambda i: (i, 0)),
            pl.BlockSpec((_BLK, 1), lambda i: (i, 0)),
            pl.BlockSpec((1, 128), lambda i: (0, 0)),
            pl.BlockSpec((128, D_OUT), lambda i: (0, 0)),
        ],
        out_specs=pl.BlockSpec((_BLK, D_OUT), lambda i: (i, 0)),
        out_shape=jax.ShapeDtypeStruct((N_PAD, D_OUT), jnp.float32),
    )(s0, s1, g1, dinv, b1, W2)


def _dense3_body(s0_ref, s1_ref, g_ref, dinv_ref, b_ref, out_ref):
    y = dinv_ref[...] * (s0_ref[...] + s1_ref[...] + g_ref[...]) + b_ref[...]
    out_ref[...] = jnp.maximum(y, 0.0)


def _dense3(s0, s1, g2, dinv, b2):
    return pl.pallas_call(
        _dense3_body,
        grid=_GRID,
        in_specs=[
            pl.BlockSpec((_BLK, D_OUT), lambda i: (i, 0)),
            pl.BlockSpec((_BLK, D_OUT), lambda i: (i, 0)),
            pl.BlockSpec((_BLK, D_OUT), lambda i: (i, 0)),
            pl.BlockSpec((_BLK, 1), lambda i: (i, 0)),
            pl.BlockSpec((1, D_OUT), lambda i: (0, 0)),
        ],
        out_specs=pl.BlockSpec((_BLK, D_OUT), lambda i: (i, 0)),
        out_shape=jax.ShapeDtypeStruct((N_PAD, D_OUT), jnp.float32),
    )(s0, s1, g2, dinv, b2)


def kernel(x, edge_index, W1, b1, W2, b2):
    src = edge_index[0]
    dst = edge_index[1]
    pad = E_PAD - N_EDGES
    src_p = jnp.concatenate([src, jnp.zeros((pad,), jnp.int32)])
    dst_p = jnp.concatenate([dst, jnp.full((pad,), N_NODES, jnp.int32)])
    x_p = jnp.pad(x, ((0, N_PAD - N_NODES), (0, 0)))
    dst_r = dst_p.reshape(NW, CHUNKS, K)
    src_f = src_p.reshape(TOTAL_CHUNKS, K)
    dst_f = dst_p.reshape(TOTAL_CHUNKS, K)
    src_e = jnp.concatenate([src_f, jnp.zeros((2, K), jnp.int32)], axis=0)
    dst_e = jnp.concatenate(
        [dst_f, jnp.full((2, K), N_NODES, jnp.int32)], axis=0)
    eidx = jnp.stack([src_e, dst_e], axis=1)

    deg_parts = _deg_kernel(dst_r)
    d0 = deg_parts[0][:, None]
    d1 = deg_parts[1][:, None]

    g1, dinv = _dense1(x_p, W1, d0, d1)
    s1 = _scatter128(g1, eidx)
    b1p = jnp.pad(b1, (0, 128 - D_HID)).reshape(1, 128)
    W2p = jnp.pad(W2, ((0, 128 - D_HID), (0, 0)))
    g2 = _dense2(s1[0], s1[1], g1, dinv, b1p, W2p)
    s2 = _scatter128(g2, eidx)
    out = _dense3(s2[0], s2[1], g2, dinv, b2.reshape(1, -1))
    return out[:N_NODES]


# speedup vs baseline: 1.0307x; 1.0307x over previous
import functools

import jax
import jax.numpy as jnp
from jax import lax
from jax.experimental import pallas as pl
from jax.experimental.pallas import tpu as pltpu
from jax.experimental.pallas import tpu_sc as plsc

N_NODES = 10000
N_EDGES = 320000
D_IN = 128
D_HID = 64
D_OUT = 128

NC = 2
NS = 16
NW = NC * NS
N_PAD = 10240
ROWS_PER_TILE = N_PAD // NS
E_PAD = NW * 10240
E_PER_W = E_PAD // NW
K = 64
CHUNKS = E_PER_W // K
CHUNKS_P = CHUNKS + 2
NBUF = 5
TOTAL_CHUNKS = E_PAD // K
CPP = TOTAL_CHUNKS // NS
N0C = 240

_MESH = dict(core_axis_name="c", subcore_axis_name="s")


@functools.partial(
    pl.kernel,
    out_type=jax.ShapeDtypeStruct((NC, N_PAD), jnp.float32),
    mesh=plsc.VectorSubcoreMesh(**_MESH),
    scratch_types=[
        pltpu.VMEM((CHUNKS, K), jnp.int32),
        pltpu.VMEM((K,), jnp.float32),
        pltpu.VMEM((ROWS_PER_TILE,), jnp.float32),
        pltpu.VMEM_SHARED((N_PAD,), jnp.float32),
        pltpu.SemaphoreType.DMA,
    ],
)
def _deg_kernel(dst_hbm, out_hbm, idx_v, ones_v, zero_v, acc_sh, sem):
    c = lax.axis_index("c")
    s = lax.axis_index("s")
    wid = s * NC + c
    for i in range(K // 16):
        ones_v[pl.ds(i * 16, 16)] = jnp.full((16,), 1.0, jnp.float32)
    for i in range(ROWS_PER_TILE // 16):
        zero_v[pl.ds(i * 16, 16)] = jnp.zeros((16,), jnp.float32)
    pltpu.sync_copy(zero_v, acc_sh.at[pl.ds(s * ROWS_PER_TILE, ROWS_PER_TILE)])
    pltpu.sync_copy(dst_hbm.at[wid], idx_v)
    plsc.subcore_barrier()

    def fire(j, carry):
        pltpu.async_copy(ones_v, acc_sh.at[idx_v.at[j]], sem, add=True)
        return carry

    lax.fori_loop(0, CHUNKS, fire, 0)

    def drain(j, carry):
        pltpu.make_async_copy(ones_v, acc_sh.at[idx_v.at[0]], sem).wait()
        return carry

    lax.fori_loop(0, CHUNKS, drain, 0)
    plsc.subcore_barrier()
    pltpu.sync_copy(acc_sh.at[pl.ds(s * ROWS_PER_TILE, ROWS_PER_TILE)],
                    out_hbm.at[c, pl.ds(s * ROWS_PER_TILE, ROWS_PER_TILE)])


def _make_scatter(D):
    @functools.partial(
        pl.kernel,
        out_type=jax.ShapeDtypeStruct((NC, N_PAD, D), jnp.float32),
        mesh=plsc.VectorSubcoreMesh(**_MESH),
        scratch_types=[
            pltpu.VMEM((NBUF, 2, K), jnp.int32),
            pltpu.VMEM((NBUF, K, D), jnp.float32),
            pltpu.VMEM_SHARED((N_PAD, D), jnp.float32),
            pltpu.SemaphoreType.DMA((NBUF,)),
            pltpu.SemaphoreType.DMA((NBUF,)),
            pltpu.SemaphoreType.DMA((NBUF,)),
        ],
    )
    def scatter(g_hbm, eidx_hbm, out_hbm, ibuf, rows, acc_sh, isem, gsem, ssem):
        c = lax.axis_index("c")
        s = lax.axis_index("s")
        start = s * CPP + c * N0C
        cnt = jnp.where(c == 0, N0C, CPP - N0C)

        def zfill(j, carry):
            for t in range(D // 16):
                rows[0, j, pl.ds(t * 16, 16)] = jnp.zeros((16,), jnp.float32)
            return carry

        lax.fori_loop(0, K, zfill, 0)
        for r in range(ROWS_PER_TILE // K):
            pltpu.sync_copy(
                rows.at[0], acc_sh.at[pl.ds(s * ROWS_PER_TILE + r * K, K)])
        plsc.subcore_barrier()

        pltpu.async_copy(eidx_hbm.at[start], ibuf.at[0], isem.at[0])
        pltpu.async_copy(eidx_hbm.at[start + 1], ibuf.at[1], isem.at[1])
        pltpu.make_async_copy(eidx_hbm.at[0], ibuf.at[0], isem.at[0]).wait()
        pltpu.async_copy(g_hbm.at[ibuf.at[0, 0]], rows.at[0], gsem.at[0])

        def step(j, b, with_swait):
            b2 = (b + 2) % NBUF
            b1 = (b + 1) % NBUF
            if with_swait:
                pltpu.make_async_copy(rows.at[b2], acc_sh.at[ibuf.at[b2, 1]],
                                      ssem.at[b2]).wait()
            pltpu.async_copy(eidx_hbm.at[start + j + 2], ibuf.at[b2],
                             isem.at[b2])
            pltpu.make_async_copy(eidx_hbm.at[0], ibuf.at[b1],
                                  isem.at[b1]).wait()
            pltpu.async_copy(g_hbm.at[ibuf.at[b1, 0]], rows.at[b1],
                             gsem.at[b1])
            pltpu.make_async_copy(g_hbm.at[ibuf.at[b, 0]], rows.at[b],
                                  gsem.at[b]).wait()
            pltpu.async_copy(rows.at[b], acc_sh.at[ibuf.at[b, 1]],
                             ssem.at[b], add=True)

        for j0 in range(NBUF):
            step(j0, j0, with_swait=(j0 >= NBUF - 2))

        def body(jj, carry):
            for b in range(NBUF):
                step(jj * NBUF + b, b, True)
            return carry

        lax.fori_loop(1, cnt // NBUF, body, 0)
        for bd in range(2, NBUF):
            pltpu.make_async_copy(rows.at[bd], acc_sh.at[ibuf.at[bd, 1]],
                                  ssem.at[bd]).wait()
        pltpu.make_async_copy(eidx_hbm.at[0], ibuf.at[1], isem.at[1]).wait()
        pltpu.make_async_copy(g_hbm.at[ibuf.at[0, 0]], rows.at[0],
                              gsem.at[0]).wait()
        plsc.subcore_barrier()
        off = s * ROWS_PER_TILE
        pltpu.sync_copy(acc_sh.at[pl.ds(off, ROWS_PER_TILE)],
                        out_hbm.at[c, pl.ds(off, ROWS_PER_TILE)])

    return scatter


_scatter128 = _make_scatter(128)


_BLK = 512
_GRID = (N_PAD // _BLK,)


def _dense1_body(x_ref, w_ref, d0_ref, d1_ref, g_ref, dinv_ref):
    deg = d0_ref[...] + d1_ref[...] + 1.0
    dinv = lax.rsqrt(deg)
    dinv_ref[...] = dinv
    h = jnp.dot(x_ref[...], w_ref[...], preferred_element_type=jnp.float32)
    g_ref[...] = jnp.concatenate(
        [h * dinv, jnp.zeros((_BLK, 128 - D_HID), jnp.float32)], axis=1)


def _dense1(x_p, W1, d0, d1):
    return pl.pallas_call(
        _dense1_body,
        grid=_GRID,
        in_specs=[
            pl.BlockSpec((_BLK, D_IN), lambda i: (i, 0)),
            pl.BlockSpec((D_IN, D_HID), lambda i: (0, 0)),
            pl.BlockSpec((_BLK, 1), lambda i: (i, 0)),
            pl.BlockSpec((_BLK, 1), lambda i: (i, 0)),
        ],
        out_specs=[
            pl.BlockSpec((_BLK, 128), lambda i: (i, 0)),
            pl.BlockSpec((_BLK, 1), lambda i: (i, 0)),
        ],
        out_shape=[
            jax.ShapeDtypeStruct((N_PAD, 128), jnp.float32),
            jax.ShapeDtypeStruct((N_PAD, 1), jnp.float32),
        ],
    )(x_p, W1, d0, d1)


def _dense2_body(s0_ref, s1_ref, g_ref, dinv_ref, b_ref, w_ref, g2_ref):
    dinv = dinv_ref[...]
    y = dinv * (s0_ref[...] + s1_ref[...] + g_ref[...]) + b_ref[...]
    y = jnp.maximum(y, 0.0)
    h2 = jnp.dot(y, w_ref[...], preferred_element_type=jnp.float32)
    g2_ref[...] = h2 * dinv


def _dense2(s0, s1, g1, dinv, b1, W2):
    return pl.pallas_call(
        _dense2_body,
        grid=_GRID,
        in_specs=[
            pl.BlockSpec((_BLK, 128), lambda i: (i, 0)),
            pl.BlockSpec((_BLK, 128), lambda i: (i, 0)),
            pl.BlockSpec((_BLK, 128), lambda i: (i, 0)),
            pl.BlockSpec((_BLK, 1), lambda i: (i, 0)),
            pl.BlockSpec((1, 128), lambda i: (0, 0)),
            pl.BlockSpec((128, D_OUT), lambda i: (0, 0)),
        ],
        out_specs=pl.BlockSpec((_BLK, D_OUT), lambda i: (i, 0)),
        out_shape=jax.ShapeDtypeStruct((N_PAD, D_OUT), jnp.float32),
    )(s0, s1, g1, dinv, b1, W2)


def _dense3_body(s0_ref, s1_ref, g_ref, dinv_ref, b_ref, out_ref):
    y = dinv_ref[...] * (s0_ref[...] + s1_ref[...] + g_ref[...]) + b_ref[...]
    out_ref[...] = jnp.maximum(y, 0.0)


def _dense3(s0, s1, g2, dinv, b2):
    return pl.pallas_call(
        _dense3_body,
        grid=_GRID,
        in_specs=[
            pl.BlockSpec((_BLK, D_OUT), lambda i: (i, 0)),
            pl.BlockSpec((_BLK, D_OUT), lambda i: (i, 0)),
            pl.BlockSpec((_BLK, D_OUT), lambda i: (i, 0)),
            pl.BlockSpec((_BLK, 1), lambda i: (i, 0)),
            pl.BlockSpec((1, D_OUT), lambda i: (0, 0)),
        ],
        out_specs=pl.BlockSpec((_BLK, D_OUT), lambda i: (i, 0)),
        out_shape=jax.ShapeDtypeStruct((N_PAD, D_OUT), jnp.float32),
    )(s0, s1, g2, dinv, b2)


def kernel(x, edge_index, W1, b1, W2, b2):
    src = edge_index[0]
    dst = edge_index[1]
    pad = E_PAD - N_EDGES
    src_p = jnp.concatenate([src, jnp.zeros((pad,), jnp.int32)])
    dst_p = jnp.concatenate([dst, jnp.full((pad,), N_NODES, jnp.int32)])
    x_p = jnp.pad(x, ((0, N_PAD - N_NODES), (0, 0)))
    dst_r = dst_p.reshape(NW, CHUNKS, K)
    src_f = src_p.reshape(TOTAL_CHUNKS, K)
    dst_f = dst_p.reshape(TOTAL_CHUNKS, K)
    src_e = jnp.concatenate([src_f, jnp.zeros((2, K), jnp.int32)], axis=0)
    dst_e = jnp.concatenate(
        [dst_f, jnp.full((2, K), N_NODES, jnp.int32)], axis=0)
    eidx = jnp.stack([src_e, dst_e], axis=1)

    deg_parts = _deg_kernel(dst_r)
    d0 = deg_parts[0][:, None]
    d1 = deg_parts[1][:, None]

    g1, dinv = _dense1(x_p, W1, d0, d1)
    s1 = _scatter128(g1, eidx)
    b1p = jnp.pad(b1, (0, 128 - D_HID)).reshape(1, 128)
    W2p = jnp.pad(W2, ((0, 128 - D_HID), (0, 0)))
    g2 = _dense2(s1[0], s1[1], g1, dinv, b1p, W2p)
    s2 = _scatter128(g2, eidx)
    out = _dense3(s2[0], s2[1], g2, dinv, b2.reshape(1, -1))
    return out[:N_NODES]


# speedup vs baseline: 1.0451x; 1.0140x over previous
import functools

import jax
import jax.numpy as jnp
from jax import lax
from jax.experimental import pallas as pl
from jax.experimental.pallas import tpu as pltpu
from jax.experimental.pallas import tpu_sc as plsc

N_NODES = 10000
N_EDGES = 320000
D_IN = 128
D_HID = 64
D_OUT = 128

NC = 2
NS = 16
NW = NC * NS
N_PAD = 10240
ROWS_PER_TILE = N_PAD // NS
E_PAD = NW * 10240
E_PER_W = E_PAD // NW
K = 80
CHUNKS = E_PER_W // K
CHUNKS_P = CHUNKS + 2
NBUF = 4
TOTAL_CHUNKS = E_PAD // K
CPP = TOTAL_CHUNKS // NS
N0C = 192

_MESH = dict(core_axis_name="c", subcore_axis_name="s")


@functools.partial(
    pl.kernel,
    out_type=jax.ShapeDtypeStruct((NC, N_PAD), jnp.float32),
    mesh=plsc.VectorSubcoreMesh(**_MESH),
    scratch_types=[
        pltpu.VMEM((CHUNKS, K), jnp.int32),
        pltpu.VMEM((K,), jnp.float32),
        pltpu.VMEM((ROWS_PER_TILE,), jnp.float32),
        pltpu.VMEM_SHARED((N_PAD,), jnp.float32),
        pltpu.SemaphoreType.DMA,
    ],
)
def _deg_kernel(dst_hbm, out_hbm, idx_v, ones_v, zero_v, acc_sh, sem):
    c = lax.axis_index("c")
    s = lax.axis_index("s")
    wid = s * NC + c
    for i in range(K // 16):
        ones_v[pl.ds(i * 16, 16)] = jnp.full((16,), 1.0, jnp.float32)
    for i in range(ROWS_PER_TILE // 16):
        zero_v[pl.ds(i * 16, 16)] = jnp.zeros((16,), jnp.float32)
    pltpu.sync_copy(zero_v, acc_sh.at[pl.ds(s * ROWS_PER_TILE, ROWS_PER_TILE)])
    pltpu.sync_copy(dst_hbm.at[wid], idx_v)
    plsc.subcore_barrier()

    def fire(j, carry):
        pltpu.async_copy(ones_v, acc_sh.at[idx_v.at[j]], sem, add=True)
        return carry

    lax.fori_loop(0, CHUNKS, fire, 0)

    def drain(j, carry):
        pltpu.make_async_copy(ones_v, acc_sh.at[idx_v.at[0]], sem).wait()
        return carry

    lax.fori_loop(0, CHUNKS, drain, 0)
    plsc.subcore_barrier()
    pltpu.sync_copy(acc_sh.at[pl.ds(s * ROWS_PER_TILE, ROWS_PER_TILE)],
                    out_hbm.at[c, pl.ds(s * ROWS_PER_TILE, ROWS_PER_TILE)])


def _make_scatter(D):
    @functools.partial(
        pl.kernel,
        out_type=jax.ShapeDtypeStruct((NC, N_PAD, D), jnp.float32),
        mesh=plsc.VectorSubcoreMesh(**_MESH),
        scratch_types=[
            pltpu.VMEM((NBUF, 2, K), jnp.int32),
            pltpu.VMEM((NBUF, K, D), jnp.float32),
            pltpu.VMEM_SHARED((N_PAD, D), jnp.float32),
            pltpu.SemaphoreType.DMA((NBUF,)),
            pltpu.SemaphoreType.DMA((NBUF,)),
            pltpu.SemaphoreType.DMA((NBUF,)),
        ],
    )
    def scatter(g_hbm, eidx_hbm, out_hbm, ibuf, rows, acc_sh, isem, gsem, ssem):
        c = lax.axis_index("c")
        s = lax.axis_index("s")
        start = s * CPP + c * N0C
        cnt = jnp.where(c == 0, N0C, CPP - N0C)

        def zfill(j, carry):
            for t in range(D // 16):
                rows[0, j, pl.ds(t * 16, 16)] = jnp.zeros((16,), jnp.float32)
            return carry

        lax.fori_loop(0, K, zfill, 0)
        for r in range(ROWS_PER_TILE // K):
            pltpu.sync_copy(
                rows.at[0], acc_sh.at[pl.ds(s * ROWS_PER_TILE + r * K, K)])
        plsc.subcore_barrier()

        pltpu.async_copy(eidx_hbm.at[start], ibuf.at[0], isem.at[0])
        pltpu.async_copy(eidx_hbm.at[start + 1], ibuf.at[1], isem.at[1])
        pltpu.make_async_copy(eidx_hbm.at[0], ibuf.at[0], isem.at[0]).wait()
        pltpu.async_copy(g_hbm.at[ibuf.at[0, 0]], rows.at[0], gsem.at[0])

        def step(j, b, with_swait):
            b2 = (b + 2) % NBUF
            b1 = (b + 1) % NBUF
            if with_swait:
                pltpu.make_async_copy(rows.at[b2], acc_sh.at[ibuf.at[b2, 1]],
                                      ssem.at[b2]).wait()
            pltpu.async_copy(eidx_hbm.at[start + j + 2], ibuf.at[b2],
                             isem.at[b2])
            pltpu.make_async_copy(eidx_hbm.at[0], ibuf.at[b1],
                                  isem.at[b1]).wait()
            pltpu.async_copy(g_hbm.at[ibuf.at[b1, 0]], rows.at[b1],
                             gsem.at[b1])
            pltpu.make_async_copy(g_hbm.at[ibuf.at[b, 0]], rows.at[b],
                                  gsem.at[b]).wait()
            pltpu.async_copy(rows.at[b], acc_sh.at[ibuf.at[b, 1]],
                             ssem.at[b], add=True)

        for j0 in range(NBUF):
            step(j0, j0, with_swait=(j0 >= NBUF - 2))

        def body(jj, carry):
            for b in range(NBUF):
                step(jj * NBUF + b, b, True)
            return carry

        lax.fori_loop(1, cnt // NBUF, body, 0)
        for bd in range(2, NBUF):
            pltpu.make_async_copy(rows.at[bd], acc_sh.at[ibuf.at[bd, 1]],
                                  ssem.at[bd]).wait()
        pltpu.make_async_copy(eidx_hbm.at[0], ibuf.at[1], isem.at[1]).wait()
        pltpu.make_async_copy(g_hbm.at[ibuf.at[0, 0]], rows.at[0],
                              gsem.at[0]).wait()
        plsc.subcore_barrier()
        off = s * ROWS_PER_TILE
        pltpu.sync_copy(acc_sh.at[pl.ds(off, ROWS_PER_TILE)],
                        out_hbm.at[c, pl.ds(off, ROWS_PER_TILE)])

    return scatter


_scatter128 = _make_scatter(128)


_BLK = 512
_GRID = (N_PAD // _BLK,)


def _dense1_body(x_ref, w_ref, d0_ref, d1_ref, g_ref, dinv_ref):
    deg = d0_ref[...] + d1_ref[...] + 1.0
    dinv = lax.rsqrt(deg)
    dinv_ref[...] = dinv
    h = jnp.dot(x_ref[...], w_ref[...], preferred_element_type=jnp.float32)
    g_ref[...] = jnp.concatenate(
        [h * dinv, jnp.zeros((_BLK, 128 - D_HID), jnp.float32)], axis=1)


def _dense1(x_p, W1, d0, d1):
    return pl.pallas_call(
        _dense1_body,
        grid=_GRID,
        in_specs=[
            pl.BlockSpec((_BLK, D_IN), lambda i: (i, 0)),
            pl.BlockSpec((D_IN, D_HID), lambda i: (0, 0)),
            pl.BlockSpec((_BLK, 1), lambda i: (i, 0)),
            pl.BlockSpec((_BLK, 1), lambda i: (i, 0)),
        ],
        out_specs=[
            pl.BlockSpec((_BLK, 128), lambda i: (i, 0)),
            pl.BlockSpec((_BLK, 1), lambda i: (i, 0)),
        ],
        out_shape=[
            jax.ShapeDtypeStruct((N_PAD, 128), jnp.float32),
            jax.ShapeDtypeStruct((N_PAD, 1), jnp.float32),
        ],
    )(x_p, W1, d0, d1)


def _dense2_body(s0_ref, s1_ref, g_ref, dinv_ref, b_ref, w_ref, g2_ref):
    dinv = dinv_ref[...]
    y = dinv * (s0_ref[...] + s1_ref[...] + g_ref[...]) + b_ref[...]
    y = jnp.maximum(y, 0.0)
    h2 = jnp.dot(y, w_ref[...], preferred_element_type=jnp.float32)
    g2_ref[...] = h2 * dinv


def _dense2(s0, s1, g1, dinv, b1, W2):
    return pl.pallas_call(
        _dense2_body,
        grid=_GRID,
        in_specs=[
            pl.BlockSpec((_BLK, 128), lambda i: (i, 0)),
            pl.BlockSpec((_BLK, 128), lambda i: (i, 0)),
            pl.BlockSpec((_BLK, 128), lambda i: (i, 0)),
            pl.BlockSpec((_BLK, 1), lambda i: (i, 0)),
            pl.BlockSpec((1, 128), lambda i: (0, 0)),
            pl.BlockSpec((128, D_OUT), lambda i: (0, 0)),
        ],
        out_specs=pl.BlockSpec((_BLK, D_OUT), lambda i: (i, 0)),
        out_shape=jax.ShapeDtypeStruct((N_PAD, D_OUT), jnp.float32),
    )(s0, s1, g1, dinv, b1, W2)


def _dense3_body(s0_ref, s1_ref, g_ref, dinv_ref, b_ref, out_ref):
    y = dinv_ref[...] * (s0_ref[...] + s1_ref[...] + g_ref[...]) + b_ref[...]
    out_ref[...] = jnp.maximum(y, 0.0)


def _dense3(s0, s1, g2, dinv, b2):
    return pl.pallas_call(
        _dense3_body,
        grid=_GRID,
        in_specs=[
            pl.BlockSpec((_BLK, D_OUT), lambda i: (i, 0)),
            pl.BlockSpec((_BLK, D_OUT), lambda i: (i, 0)),
            pl.BlockSpec((_BLK, D_OUT), lambda i: (i, 0)),
            pl.BlockSpec((_BLK, 1), lambda i: (i, 0)),
            pl.BlockSpec((1, D_OUT), lambda i: (0, 0)),
        ],
        out_specs=pl.BlockSpec((_BLK, D_OUT), lambda i: (i, 0)),
        out_shape=jax.ShapeDtypeStruct((N_PAD, D_OUT), jnp.float32),
    )(s0, s1, g2, dinv, b2)


def kernel(x, edge_index, W1, b1, W2, b2):
    src = edge_index[0]
    dst = edge_index[1]
    pad = E_PAD - N_EDGES
    src_p = jnp.concatenate([src, jnp.zeros((pad,), jnp.int32)])
    dst_p = jnp.concatenate([dst, jnp.full((pad,), N_NODES, jnp.int32)])
    x_p = jnp.pad(x, ((0, N_PAD - N_NODES), (0, 0)))
    dst_r = dst_p.reshape(NW, CHUNKS, K)
    src_f = src_p.reshape(TOTAL_CHUNKS, K)
    dst_f = dst_p.reshape(TOTAL_CHUNKS, K)
    src_e = jnp.concatenate([src_f, jnp.zeros((2, K), jnp.int32)], axis=0)
    dst_e = jnp.concatenate(
        [dst_f, jnp.full((2, K), N_NODES, jnp.int32)], axis=0)
    eidx = jnp.stack([src_e, dst_e], axis=1)

    deg_parts = _deg_kernel(dst_r)
    d0 = deg_parts[0][:, None]
    d1 = deg_parts[1][:, None]

    g1, dinv = _dense1(x_p, W1, d0, d1)
    s1 = _scatter128(g1, eidx)
    b1p = jnp.pad(b1, (0, 128 - D_HID)).reshape(1, 128)
    W2p = jnp.pad(W2, ((0, 128 - D_HID), (0, 0)))
    g2 = _dense2(s1[0], s1[1], g1, dinv, b1p, W2p)
    s2 = _scatter128(g2, eidx)
    out = _dense3(s2[0], s2[1], g2, dinv, b2.reshape(1, -1))
    return out[:N_NODES]
